# baseline (speedup 1.0000x reference)
import functools

import jax
import jax.numpy as jnp
from jax import lax
from jax.experimental import pallas as pl
from jax.experimental.pallas import tpu as pltpu
from jax.experimental.pallas import tpu_sc as plsc

N = 10000
E = 320000
NC = 2
NS = 16
NW = NC * NS
N_PAD = 10240
BATCH = 128
CHUNK = BATCH
NCH = 80
E_PER_TILE = NCH * CHUNK
E_PAD = NW * E_PER_TILE
IDX_ROWS = E_PAD // BATCH
JROWS = NCH
KB = 2
DEG_NCH = IDX_ROWS // NW // KB
ROWS_PER_TILE = N_PAD // NS
DW = 16

_sc_mesh = plsc.VectorSubcoreMesh(core_axis_name="c", subcore_axis_name="s")
_sc_params = pltpu.CompilerParams(use_tc_tiling_on_sc=False)
_sc_params_nl = pltpu.CompilerParams(use_tc_tiling_on_sc=False,
                                     needs_layout_passes=False)


@functools.partial(
    pl.kernel,
    out_type=jax.ShapeDtypeStruct((NW, N_PAD), jnp.float32),
    mesh=_sc_mesh,
    compiler_params=_sc_params_nl,
    scratch_types=[
        pltpu.VMEM((DEG_NCH * KB, BATCH), jnp.int32),
        pltpu.VMEM((N_PAD,), jnp.float32),
    ],
)
def _sc_degree(colidx_hbm, out_hbm, colv, deg):
    c = lax.axis_index("c")
    s = lax.axis_index("s")
    w = c * NS + s
    pltpu.sync_copy(colidx_hbm.at[pl.ds(w * DEG_NCH * KB, DEG_NCH * KB)], colv)
    ones16 = jnp.ones((16,), jnp.float32)

    def zbody(i, carry):
        deg[pl.ds(i * 16, 16)] = jnp.zeros((16,), jnp.float32)
        return carry

    lax.fori_loop(0, N_PAD // 16, zbody, 0)

    def body(i, carry):
        for l in range(BATCH // 16):
            idx = colv[i, pl.ds(l * 16, 16)]
            plsc.addupdate_scatter(deg, [idx], ones16)
        return carry

    lax.fori_loop(0, DEG_NCH * KB, body, 0)
    pltpu.sync_copy(deg, out_hbm.at[w])


@functools.partial(
    pl.kernel,
    out_type=jax.ShapeDtypeStruct((NC, N, 64), jnp.float32),
    mesh=_sc_mesh,
    compiler_params=_sc_params,
    scratch_types=[
        pltpu.VMEM((JROWS, BATCH), jnp.int32),
        pltpu.VMEM((JROWS, BATCH), jnp.int32),
        pltpu.VMEM((CHUNK, 64), jnp.float32),
        pltpu.VMEM((CHUNK, 64), jnp.float32),
        pltpu.VMEM_SHARED((N_PAD, 64), jnp.float32),
        pltpu.VMEM_SHARED((N, 64), jnp.float32),
        pltpu.SemaphoreType.DMA,
        pltpu.SemaphoreType.DMA,
        pltpu.SemaphoreType.DMA,
        pltpu.SemaphoreType.DMA,
    ],
)
def _sc_edge_scatter(g_hbm, pkidx_hbm, zeros_hbm, out_hbm,
                     rowv, colv, buf0, buf1, acc_sh, g_sh, sg0, sg1, ss0, ss1):
    c = lax.axis_index("c")
    s = lax.axis_index("s")
    w = c * NS + s
    pltpu.sync_copy(pkidx_hbm.at[pl.ds(w * JROWS, JROWS)], rowv)
    pltpu.sync_copy(
        zeros_hbm,
        acc_sh.at[pl.ds(s * ROWS_PER_TILE, ROWS_PER_TILE)],
    )
    pltpu.sync_copy(
        g_hbm.at[pl.ds(s * (N // NS), N // NS)],
        g_sh.at[pl.ds(s * (N // NS), N // NS)],
    )

    def ubody(i, carry):
        rrow = rowv.at[i]
        rcol = colv.at[i]
        for l in range(BATCH // 16):
            v = rrow[pl.ds(l * 16, 16)]
            rcol[pl.ds(l * 16, 16)] = lax.shift_right_logical(v, 14)
            rrow[pl.ds(l * 16, 16)] = lax.bitwise_and(v, 16383)
        return carry

    lax.fori_loop(0, JROWS, ubody, 0)
    plsc.subcore_barrier()

    def fire_gather(chunk, buf, sem):
        pltpu.async_copy(g_sh.at[rowv.at[chunk]], buf, sem)

    def fire_scatter(chunk, buf, sem):
        pltpu.async_copy(buf, acc_sh.at[colv.at[chunk]], sem, add=True)

    def wait_chunk(buf, sem):
        pltpu.make_async_copy(zeros_hbm.at[pl.ds(0, CHUNK)], buf, sem).wait()

    fire_gather(0, buf0, sg0)
    fire_gather(1, buf1, sg1)

    def body(k, carry):
        c0 = 2 * k
        wait_chunk(buf0, sg0)
        fire_scatter(c0, buf0, ss0)
        wait_chunk(buf1, sg1)
        fire_scatter(c0 + 1, buf1, ss1)
        wait_chunk(buf0, ss0)
        fire_gather(c0 + 2, buf0, sg0)
        wait_chunk(buf1, ss1)
        fire_gather(c0 + 3, buf1, sg1)
        return carry

    lax.fori_loop(0, NCH // 2 - 1, body, 0)
    wait_chunk(buf0, sg0)
    fire_scatter(NCH - 2, buf0, ss0)
    wait_chunk(buf1, sg1)
    fire_scatter(NCH - 1, buf1, ss1)
    wait_chunk(buf0, ss0)
    wait_chunk(buf1, ss1)
    plsc.subcore_barrier()
    pltpu.sync_copy(
        acc_sh.at[pl.ds(s * (N // NS), N // NS)],
        out_hbm.at[c, pl.ds(s * (N // NS), N // NS)],
    )


def _tc_mm_body(x_ref, w_ref, o_ref):
    o_ref[...] = jnp.dot(x_ref[...], w_ref[...],
                         preferred_element_type=jnp.float32)


def _tc_mm(x, w):
    return pl.pallas_call(
        _tc_mm_body,
        out_shape=jax.ShapeDtypeStruct((x.shape[0], w.shape[1]), jnp.float32),
    )(x, w)


def _tc_norm_body(degp_ref, h_ref, g_ref, dinv_ref):
    deg = jnp.sum(degp_ref[...], axis=1, keepdims=True) + 1.0
    dinv = lax.rsqrt(deg)[:N]
    dinv_ref[...] = dinv
    g_ref[...] = h_ref[...] * dinv


def _tc_norm(degp, h):
    return pl.pallas_call(
        _tc_norm_body,
        out_shape=(
            jax.ShapeDtypeStruct((N, 64), jnp.float32),
            jax.ShapeDtypeStruct((N, 1), jnp.float32),
        ),
    )(degp, h)


def _tc_mid_body(sp_ref, g1_ref, dinv_ref, b1_ref, w2_ref, g2_ref):
    s = sp_ref[0, :N] + sp_ref[1, :N]
    dinv = dinv_ref[...]
    z = dinv * (s + g1_ref[...]) + b1_ref[...]
    z = jnp.maximum(z, 0.0)
    h2 = jnp.dot(z, w2_ref[...], preferred_element_type=jnp.float32)
    g2_ref[...] = h2 * dinv


def _tc_mid(sp, g1, dinv, b1, w2):
    return pl.pallas_call(
        _tc_mid_body,
        out_shape=jax.ShapeDtypeStruct((N, 64), jnp.float32),
    )(sp, g1, dinv, b1, w2)


def _tc_final_body(sp_ref, g2_ref, dinv_ref, b2_ref, o_ref):
    s = sp_ref[0, :N] + sp_ref[1, :N]
    o_ref[...] = dinv_ref[...] * (s + g2_ref[...]) + b2_ref[...]


def _tc_final(sp, g2, dinv, b2):
    return pl.pallas_call(
        _tc_final_body,
        out_shape=jax.ShapeDtypeStruct((N, 64), jnp.float32),
    )(sp, g2, dinv, b2)


def kernel(data, edge_idx, W1, b1, W2, b2):
    row = edge_idx[0].astype(jnp.int32)
    col = edge_idx[1].astype(jnp.int32)
    pad = E_PAD - E
    row_p = jnp.concatenate([row, jnp.zeros((pad,), jnp.int32)])
    col_p = jnp.concatenate(
        [col, N + (jnp.arange(pad, dtype=jnp.int32) % (N_PAD - N))])
    colidx = col_p.reshape(IDX_ROWS, BATCH)
    pkidx = (row_p | (col_p << 14)).reshape(IDX_ROWS, BATCH)

    zeros_acc = jnp.zeros((ROWS_PER_TILE, 64), jnp.float32)

    degp = _sc_degree(colidx)
    h1 = _tc_mm(data, W1)
    g1, dinv = _tc_norm(degp.T, h1)
    s1p = _sc_edge_scatter(g1, pkidx, zeros_acc)
    g2 = _tc_mid(s1p, g1, dinv, b1.reshape(1, 64), W2)
    s2p = _sc_edge_scatter(g2, pkidx, zeros_acc)
    out = _tc_final(s2p, g2, dinv, b2.reshape(1, 64))
    return out
